# Initial kernel scaffold; baseline (speedup 1.0000x reference)
import jax
import jax.numpy as jnp
from jax.experimental import pallas as pl


def kernel(x, edge_index, edge_attr):
    raise NotImplementedError("write your pallas kernel here")


# speedup vs baseline: 2.6985x; 2.6985x over previous
import functools

import jax
import jax.numpy as jnp
from jax import lax
from jax.experimental import pallas as pl
from jax.experimental.pallas import tpu as pltpu
from jax.experimental.pallas import tpu_sc as plsc

N_NODES = 10000
N_EDGES = 320000
NPAD = 10240
NC = 2
NS = 16
NW = NC * NS
E_PER_TILE = N_EDGES // NW
CHUNK = NPAD // NS
LANES = 16


def _sc_body(x0_hbm, src_hbm, dst_hbm, attr_hbm, out_hbm,
             x0_v, src_v, dst_v, attr_v, acc_v, red_v, out_v, shared):
    c = lax.axis_index("c")
    s = lax.axis_index("s")
    wid = c * NS + s
    base = wid * E_PER_TILE

    pltpu.sync_copy(x0_hbm, x0_v)
    pltpu.sync_copy(src_hbm.at[pl.ds(base, E_PER_TILE)], src_v)
    pltpu.sync_copy(dst_hbm.at[pl.ds(base, E_PER_TILE)], dst_v)
    pltpu.sync_copy(attr_hbm.at[pl.ds(base * 4, E_PER_TILE * 4)], attr_v)

    zeros16 = jnp.zeros((LANES,), jnp.float32)

    def zbody(j, carry):
        off = j * LANES
        acc_v[pl.ds(off, LANES)] = zeros16
        return carry

    lax.fori_loop(jnp.int32(0), jnp.int32(NPAD // LANES), zbody, None)

    lane4 = lax.iota(jnp.int32, 16) * 4

    def ebody(j, carry):
        off = j * LANES
        srcv = src_v[pl.ds(off, LANES)]
        dstv = dst_v[pl.ds(off, LANES)]
        xs = plsc.load_gather(x0_v, [srcv])
        xd = plsc.load_gather(x0_v, [dstv])
        eids4 = lane4 + off * 4
        w0 = plsc.load_gather(attr_v, [eids4])
        w1 = plsc.load_gather(attr_v, [eids4 + 1])
        contrib = (xs - xd) * (w0 + w1)
        plsc.addupdate_scatter(acc_v, [dstv], contrib)
        return carry

    lax.fori_loop(jnp.int32(0), jnp.int32(E_PER_TILE // LANES), ebody, None)

    pltpu.sync_copy(acc_v, shared.at[s])
    plsc.subcore_barrier()

    nbase = s * CHUNK
    for r in range(NS):
        pltpu.sync_copy(shared.at[jnp.int32(r), pl.ds(nbase, CHUNK)],
                        red_v.at[jnp.int32(r)])

    def rbody(j, carry):
        off = j * LANES
        a = red_v[jnp.int32(0), pl.ds(off, LANES)]
        for r in range(1, NS):
            a = a + red_v[jnp.int32(r), pl.ds(off, LANES)]
        out_v[pl.ds(off, LANES)] = a
        return carry

    lax.fori_loop(jnp.int32(0), jnp.int32(CHUNK // LANES), rbody, None)
    pltpu.sync_copy(out_v, out_hbm.at[c, pl.ds(nbase, CHUNK)])


@jax.jit
def _sc_call(x0, src, dst, attr):
    mesh = plsc.VectorSubcoreMesh(core_axis_name="c", subcore_axis_name="s")
    return pl.kernel(
        _sc_body,
        out_type=jax.ShapeDtypeStruct((NC, NPAD), jnp.float32),
        mesh=mesh,
        compiler_params=pltpu.CompilerParams(
            needs_layout_passes=False, use_tc_tiling_on_sc=False),
        scratch_types=[
            pltpu.VMEM((N_NODES,), jnp.float32),
            pltpu.VMEM((E_PER_TILE,), jnp.int32),
            pltpu.VMEM((E_PER_TILE,), jnp.int32),
            pltpu.VMEM((E_PER_TILE * 4,), jnp.float32),
            pltpu.VMEM((NPAD,), jnp.float32),
            pltpu.VMEM((NS, CHUNK), jnp.float32),
            pltpu.VMEM((CHUNK,), jnp.float32),
            pltpu.VMEM_SHARED((NS, NPAD), jnp.float32),
        ],
    )(x0, src, dst, attr)


def _combine_body(p_ref, o_ref):
    o_ref[...] = p_ref[0, :] + p_ref[1, :]


@jax.jit
def _combine(partials):
    return pl.pallas_call(
        _combine_body,
        out_shape=jax.ShapeDtypeStruct((NPAD,), jnp.float32),
    )(partials)


def kernel(x, edge_index, edge_attr):
    x0 = x[:, 0]
    ei = edge_index.astype(jnp.int32)
    partials = _sc_call(x0, ei[0], ei[1], edge_attr.reshape(-1))
    return _combine(partials)[:N_NODES]
